# Initial kernel scaffold; baseline (speedup 1.0000x reference)
import jax
import jax.numpy as jnp
from jax.experimental import pallas as pl


def kernel(x, edge_index, W1_l, b1, W1_r, W2_l, b2, W2_r, Wa, ba):
    raise NotImplementedError("write your pallas kernel here")


# speedup vs baseline: 1.0135x; 1.0135x over previous
import functools

import jax
import jax.numpy as jnp
from jax.experimental import pallas as pl
from jax.experimental.pallas import tpu as pltpu

N = 10000
E = 320000
D_IN = 128
D_HID = 128
D_OUT = 64

_ROWS = 1000


def _dense1_body(agg_ref, cnt_ref, x_ref, wl_ref, b_ref, wr_ref, o_ref):
    cnt = jnp.clip(cnt_ref[...], 1.0, None)
    agg = agg_ref[...] / cnt
    acc = jax.lax.dot(agg, wl_ref[...], precision=jax.lax.Precision.HIGHEST,
                      preferred_element_type=jnp.float32)
    acc += jax.lax.dot(x_ref[...], wr_ref[...], precision=jax.lax.Precision.HIGHEST,
                       preferred_element_type=jnp.float32)
    o_ref[...] = jnp.maximum(acc + b_ref[...], 0.0)


def _dense1(agg, cnt, x, wlT, b, wrT):
    return pl.pallas_call(
        _dense1_body,
        grid=(N // _ROWS,),
        in_specs=[
            pl.BlockSpec((_ROWS, D_IN), lambda i: (i, 0)),
            pl.BlockSpec((_ROWS, 1), lambda i: (i, 0)),
            pl.BlockSpec((_ROWS, D_IN), lambda i: (i, 0)),
            pl.BlockSpec((D_IN, D_HID), lambda i: (0, 0)),
            pl.BlockSpec((1, D_HID), lambda i: (0, 0)),
            pl.BlockSpec((D_IN, D_HID), lambda i: (0, 0)),
        ],
        out_specs=pl.BlockSpec((_ROWS, D_HID), lambda i: (i, 0)),
        out_shape=jax.ShapeDtypeStruct((N, D_HID), jnp.float32),
    )(agg, cnt, x, wlT, b, wrT)


def _dense2_body(agg_ref, cnt_ref, h_ref, wl_ref, b_ref, wr_ref, wa_ref, ba_ref, o_ref):
    cnt = jnp.clip(cnt_ref[...], 1.0, None)
    agg = agg_ref[...] / cnt
    acc = jax.lax.dot(agg, wl_ref[...], precision=jax.lax.Precision.HIGHEST,
                      preferred_element_type=jnp.float32)
    acc += jax.lax.dot(h_ref[...], wr_ref[...], precision=jax.lax.Precision.HIGHEST,
                       preferred_element_type=jnp.float32)
    h2 = jnp.maximum(acc + b_ref[...], 0.0)
    o_ref[...] = jax.lax.dot(h2, wa_ref[...], precision=jax.lax.Precision.HIGHEST,
                             preferred_element_type=jnp.float32) + ba_ref[...]


def _dense2(agg, cnt, h, wlT, b, wrT, waT, ba):
    return pl.pallas_call(
        _dense2_body,
        grid=(N // _ROWS,),
        in_specs=[
            pl.BlockSpec((_ROWS, D_HID), lambda i: (i, 0)),
            pl.BlockSpec((_ROWS, 1), lambda i: (i, 0)),
            pl.BlockSpec((_ROWS, D_HID), lambda i: (i, 0)),
            pl.BlockSpec((D_HID, D_HID), lambda i: (0, 0)),
            pl.BlockSpec((1, D_HID), lambda i: (0, 0)),
            pl.BlockSpec((D_HID, D_HID), lambda i: (0, 0)),
            pl.BlockSpec((D_HID, D_OUT), lambda i: (0, 0)),
            pl.BlockSpec((1, D_OUT), lambda i: (0, 0)),
        ],
        out_specs=pl.BlockSpec((_ROWS, D_OUT), lambda i: (i, 0)),
        out_shape=jax.ShapeDtypeStruct((N, D_OUT), jnp.float32),
    )(agg, cnt, h, wlT, b, wrT, waT, ba)


def _edge_agg(x, src, dst):
    msg = jnp.take(x, src, axis=0)
    agg = jax.ops.segment_sum(msg, dst, num_segments=N)
    cnt = jax.ops.segment_sum(jnp.ones((E,), jnp.float32), dst, num_segments=N)
    return agg, cnt[:, None]


def kernel(x, edge_index, W1_l, b1, W1_r, W2_l, b2, W2_r, Wa, ba):
    src = edge_index[0].astype(jnp.int32)
    dst = edge_index[1].astype(jnp.int32)
    agg1, cnt = _edge_agg(x, src, dst)
    h1 = _dense1(agg1, cnt, x, W1_l.T, b1[None, :], W1_r.T)
    agg2, _ = _edge_agg(h1, src, dst)
    return _dense2(agg2, cnt, h1, W2_l.T, b2[None, :], W2_r.T, Wa.T, ba[None, :])


# speedup vs baseline: 1.0885x; 1.0740x over previous
import functools

import jax
import jax.numpy as jnp
from jax import lax
from jax.experimental import pallas as pl
from jax.experimental.pallas import tpu as pltpu
from jax.experimental.pallas import tpu_sc as plsc

N = 10000
E = 320000
D_IN = 128
D_HID = 128
D_OUT = 64

_NP = 10240
_NB = 32
_BKN = _NP // _NB
_CAP = E
_CH = 80
_C = 512
_NCH = E // _C


def _rank_body(dst_ref, tri_ref, pos_ref, cnt_ref, carry):
    c = pl.program_id(0)

    @pl.when(c == 0)
    def _():
        carry[...] = jnp.zeros_like(carry)

    d = dst_ref[0, 0]
    b = d // _BKN
    oh = (b[:, None] == lax.broadcasted_iota(jnp.int32, (_C, _NB), 1)
          ).astype(jnp.float32)
    pr = jax.lax.dot(tri_ref[...], oh, precision=jax.lax.Precision.HIGHEST,
                     preferred_element_type=jnp.float32)
    rank = jnp.sum(pr * oh, axis=1)
    pg = jnp.sum(carry[0][None, :] * oh, axis=1)
    pos_ref[...] = (b * _CAP + (rank + pg).astype(jnp.int32))[None, None]
    newc = carry[...] + jnp.sum(oh, axis=0)[None]
    carry[...] = newc
    cnt_ref[...] = newc


def _rank(dst2d, tri):
    return pl.pallas_call(
        _rank_body,
        grid=(_NCH,),
        in_specs=[
            pl.BlockSpec((1, 1, _C), lambda i: (i, 0, 0)),
            pl.BlockSpec((_C, _C), lambda i: (0, 0)),
        ],
        out_specs=[
            pl.BlockSpec((1, 1, _C), lambda i: (i, 0, 0)),
            pl.BlockSpec((1, _NB), lambda i: (0, 0)),
        ],
        out_shape=[
            jax.ShapeDtypeStruct((_NCH, 1, _C), jnp.int32),
            jax.ShapeDtypeStruct((1, _NB), jnp.float32),
        ],
        scratch_shapes=[pltpu.VMEM((1, _NB), jnp.float32)],
    )(dst2d, tri)


_mesh = plsc.VectorSubcoreMesh(core_axis_name="c", subcore_axis_name="s")
_EPT = E // _NB


@functools.partial(
    pl.kernel,
    out_type=(jax.ShapeDtypeStruct((_NB * _CAP,), jnp.int32),
              jax.ShapeDtypeStruct((_NB * _CAP,), jnp.int32)),
    mesh=_mesh,
    scratch_types=[
        pltpu.VMEM((_CH,), jnp.int32),
        pltpu.VMEM((_CH,), jnp.int32),
        pltpu.VMEM((_CH,), jnp.int32),
    ],
)
def _group(src_hbm, dst_hbm, pos_hbm, srcg_hbm, dstg_hbm, srcv, dstv, posv):
    w = lax.axis_index("c") * 16 + lax.axis_index("s")
    base = w * _EPT

    def chunk(c, carry):
        off = base + c * _CH
        pltpu.sync_copy(src_hbm.at[pl.ds(off, _CH)], srcv)
        pltpu.sync_copy(dst_hbm.at[pl.ds(off, _CH)], dstv)
        pltpu.sync_copy(pos_hbm.at[pl.ds(off, _CH)], posv)
        pltpu.sync_copy(srcv, srcg_hbm.at[posv])
        pltpu.sync_copy(dstv, dstg_hbm.at[posv])
        return carry

    lax.fori_loop(0, _EPT // _CH, chunk, 0)


@functools.partial(
    pl.kernel,
    out_type=(jax.ShapeDtypeStruct((_NP, D_IN), jnp.float32),
              jax.ShapeDtypeStruct((_NP, 16), jnp.float32)),
    mesh=_mesh,
    scratch_types=[
        pltpu.VMEM((_BKN + 8, D_IN), jnp.float32),
        pltpu.VMEM((_BKN + 8, 16), jnp.float32),
        pltpu.VMEM((_CH,), jnp.int32),
        pltpu.VMEM((_CH + 16,), jnp.int32),
        pltpu.VMEM((_CH, D_IN), jnp.float32),
        pltpu.VMEM((_NB + 16,), jnp.int32),
        pltpu.SemaphoreType.DMA,
    ],
)
def _agg(x_hbm, srcg_hbm, dstg_hbm, cnts_hbm, zacc_hbm, zcnt_hbm,
         agg_hbm, cnt_hbm,
         acc, cntacc, srcv, dstv, rows, cntv, sem):
    w = lax.axis_index("c") * 16 + lax.axis_index("s")
    lo = w * _BKN
    base = w * _CAP

    pltpu.sync_copy(zacc_hbm, acc)
    pltpu.sync_copy(zcnt_hbm, cntacc)
    pltpu.sync_copy(cnts_hbm, cntv)
    cb = cntv[pl.ds(w, 16)][0]
    nch = (cb + (_CH - 1)) // _CH

    def chunk(c, carry):
        off = base + c * _CH
        pltpu.sync_copy(srcg_hbm.at[pl.ds(off, _CH)], srcv)
        pltpu.sync_copy(dstg_hbm.at[pl.ds(off, _CH)], dstv.at[pl.ds(0, _CH)])
        for j in range(_CH // 16):
            sl = pl.ds(j * 16, 16)
            srcv[sl] = jnp.clip(srcv[sl], 0, _NP - 1)
        pltpu.async_copy(x_hbm.at[srcv], rows, sem).wait()
        ne = jnp.minimum(cb - c * _CH, _CH)

        def edge(i, carry2):
            ld = jnp.clip(dstv[pl.ds(i, 16)][0] - lo, 0, _BKN)
            cntacc[ld, :] = cntacc[ld, :] + 1.0
            for v in range(D_IN // 16):
                sl = pl.ds(v * 16, 16)
                acc[ld, sl] = acc[ld, sl] + rows[i, sl]
            return carry2

        lax.fori_loop(0, ne, edge, 0)
        return carry

    lax.fori_loop(0, nch, chunk, 0)

    pltpu.sync_copy(acc.at[pl.ds(0, _BKN)], agg_hbm.at[pl.ds(lo, _BKN)])
    pltpu.sync_copy(cntacc.at[pl.ds(0, _BKN)], cnt_hbm.at[pl.ds(lo, _BKN)])


_R1 = 1024
_R2 = 1000


def _dense1_body(agg_ref, c_ref, x_ref, wl_ref, b_ref, wr_ref, o_ref):
    cnt = jnp.clip(c_ref[:, 0:1], 1.0, None)
    agg = agg_ref[...] / cnt
    acc = jax.lax.dot(agg, wl_ref[...], precision=jax.lax.Precision.HIGHEST,
                      preferred_element_type=jnp.float32)
    acc += jax.lax.dot(x_ref[...], wr_ref[...], precision=jax.lax.Precision.HIGHEST,
                       preferred_element_type=jnp.float32)
    o_ref[...] = jnp.maximum(acc + b_ref[...], 0.0)


def _dense1(agg, cnt, x, wlT, b, wrT):
    return pl.pallas_call(
        _dense1_body,
        grid=(_NP // _R1,),
        in_specs=[
            pl.BlockSpec((_R1, D_IN), lambda i: (i, 0)),
            pl.BlockSpec((_R1, 16), lambda i: (i, 0)),
            pl.BlockSpec((_R1, D_IN), lambda i: (i, 0)),
            pl.BlockSpec((D_IN, D_HID), lambda i: (0, 0)),
            pl.BlockSpec((1, D_HID), lambda i: (0, 0)),
            pl.BlockSpec((D_IN, D_HID), lambda i: (0, 0)),
        ],
        out_specs=pl.BlockSpec((_R1, D_HID), lambda i: (i, 0)),
        out_shape=jax.ShapeDtypeStruct((_NP, D_HID), jnp.float32),
    )(agg, cnt, x, wlT, b, wrT)


def _dense2_body(p_ref, c_ref, h_ref, wl_ref, b_ref, wr_ref, wa_ref, ba_ref,
                 o_ref):
    cnt = jnp.clip(c_ref[:, 0:1], 1.0, None)
    agg = p_ref[...] / cnt
    acc = jax.lax.dot(agg, wl_ref[...], precision=jax.lax.Precision.HIGHEST,
                      preferred_element_type=jnp.float32)
    acc += jax.lax.dot(h_ref[...], wr_ref[...], precision=jax.lax.Precision.HIGHEST,
                       preferred_element_type=jnp.float32)
    h2 = jnp.maximum(acc + b_ref[...], 0.0)
    o_ref[...] = jax.lax.dot(h2, wa_ref[...], precision=jax.lax.Precision.HIGHEST,
                             preferred_element_type=jnp.float32) + ba_ref[...]


def _dense2(agg, cnt, h, wlT, b, wrT, waT, ba):
    return pl.pallas_call(
        _dense2_body,
        grid=(N // _R2,),
        in_specs=[
            pl.BlockSpec((_R2, D_HID), lambda i: (i, 0)),
            pl.BlockSpec((_R2, 16), lambda i: (i, 0)),
            pl.BlockSpec((_R2, D_HID), lambda i: (i, 0)),
            pl.BlockSpec((D_HID, D_HID), lambda i: (0, 0)),
            pl.BlockSpec((1, D_HID), lambda i: (0, 0)),
            pl.BlockSpec((D_HID, D_HID), lambda i: (0, 0)),
            pl.BlockSpec((D_HID, D_OUT), lambda i: (0, 0)),
            pl.BlockSpec((1, D_OUT), lambda i: (0, 0)),
        ],
        out_specs=pl.BlockSpec((_R2, D_OUT), lambda i: (i, 0)),
        out_shape=jax.ShapeDtypeStruct((N, D_OUT), jnp.float32),
    )(agg, cnt, h, wlT, b, wrT, waT, ba)


def kernel(x, edge_index, W1_l, b1, W1_r, W2_l, b2, W2_r, Wa, ba):
    src = edge_index[0].astype(jnp.int32)
    dst = edge_index[1].astype(jnp.int32)

    tri = jnp.tril(jnp.ones((_C, _C), jnp.float32), -1)
    pos2d, cntf = _rank(dst.reshape(_NCH, 1, _C), tri)
    pos = pos2d.reshape(E)
    cnts = jnp.pad(cntf[0].astype(jnp.int32), (0, 16))

    srcg, dstg = _group(src, dst, pos)

    xpad = jnp.pad(x, ((0, _NP - N), (0, 0)))
    zacc = jnp.zeros((_BKN + 8, D_IN), jnp.float32)
    zcnt = jnp.zeros((_BKN + 8, 16), jnp.float32)

    agg1, cnt = _agg(xpad, srcg, dstg, cnts, zacc, zcnt)
    h1 = _dense1(agg1, cnt, xpad, W1_l.T, b1[None, :], W1_r.T)
    agg2, _ = _agg(h1, srcg, dstg, cnts, zacc, zcnt)
    return _dense2(agg2, cnt, h1, W2_l.T, b2[None, :], W2_r.T, Wa.T, ba[None, :])


# speedup vs baseline: 1.3987x; 1.2850x over previous
import functools

import jax
import jax.numpy as jnp
from jax import lax
from jax.experimental import pallas as pl
from jax.experimental.pallas import tpu as pltpu
from jax.experimental.pallas import tpu_sc as plsc

N = 10000
E = 320000
D_IN = 128
D_HID = 128
D_OUT = 64

_NP = 10240
_NB = 32
_BKN = _NP // _NB
_CAP = E
_CH = 80
_C = 512
_NCH = E // _C


def _rank_body(dst_ref, tri_ref, pos_ref, cnt_ref, carry):
    c = pl.program_id(0)

    @pl.when(c == 0)
    def _():
        carry[...] = jnp.zeros_like(carry)

    for r in range(8):
        d = dst_ref[r]
        b = d // _BKN
        real = (c * 8 + r < _NCH).astype(jnp.float32)
        oh = (b[:, None] == lax.broadcasted_iota(jnp.int32, (_C, _NB), 1)
              ).astype(jnp.float32) * real
        pr = jax.lax.dot(tri_ref[...], oh, precision=jax.lax.Precision.HIGHEST,
                         preferred_element_type=jnp.float32)
        rank = jnp.sum(pr * oh, axis=1)
        pg = jnp.sum(carry[0][None, :] * oh, axis=1)
        pos_ref[r, :] = b * _CAP + (rank + pg).astype(jnp.int32)
        carry[...] = carry[...] + jnp.sum(oh, axis=0)[None]
    cnt_ref[...] = carry[...]


_NCHP = 640


def _rank(dst2d, tri):
    return pl.pallas_call(
        _rank_body,
        grid=(_NCHP // 8,),
        in_specs=[
            pl.BlockSpec((8, _C), lambda i: (i, 0)),
            pl.BlockSpec((_C, _C), lambda i: (0, 0)),
        ],
        out_specs=[
            pl.BlockSpec((8, _C), lambda i: (i, 0)),
            pl.BlockSpec((1, _NB), lambda i: (0, 0)),
        ],
        out_shape=[
            jax.ShapeDtypeStruct((_NCHP, _C), jnp.int32),
            jax.ShapeDtypeStruct((1, _NB), jnp.float32),
        ],
        scratch_shapes=[pltpu.VMEM((1, _NB), jnp.float32)],
    )(dst2d, tri)


_mesh = plsc.VectorSubcoreMesh(core_axis_name="c", subcore_axis_name="s")
_EPT = E // _NB


@functools.partial(
    pl.kernel,
    out_type=jax.ShapeDtypeStruct((_NB * _CAP,), jnp.int32),
    mesh=_mesh,
    scratch_types=[
        pltpu.VMEM((_CH,), jnp.int32),
        pltpu.VMEM((_CH,), jnp.int32),
        pltpu.VMEM((_CH,), jnp.int32),
        pltpu.VMEM((_CH,), jnp.int32),
    ],
)
def _group(src_hbm, dst_hbm, pos_hbm, pkg_hbm, srcv, dstv, posv, pkv):
    w = lax.axis_index("c") * 16 + lax.axis_index("s")
    base = w * _EPT

    def chunk(c, carry):
        off = base + c * _CH
        pltpu.sync_copy(src_hbm.at[pl.ds(off, _CH)], srcv)
        pltpu.sync_copy(dst_hbm.at[pl.ds(off, _CH)], dstv)
        pltpu.sync_copy(pos_hbm.at[pl.ds(off, _CH)], posv)
        for j in range(_CH // 16):
            sl = pl.ds(j * 16, 16)
            pkv[sl] = (srcv[sl] << 14) | dstv[sl]
        pltpu.sync_copy(pkv, pkg_hbm.at[posv])
        return carry

    lax.fori_loop(0, _EPT // _CH, chunk, 0)


@functools.partial(
    pl.kernel,
    out_type=(jax.ShapeDtypeStruct((_NP, D_IN), jnp.float32),
              jax.ShapeDtypeStruct((_NP, 16), jnp.float32)),
    mesh=_mesh,
    scratch_types=[
        pltpu.VMEM((_BKN + 8, D_IN), jnp.float32),
        pltpu.VMEM((_BKN + 8, 16), jnp.float32),
        pltpu.VMEM((_CH,), jnp.int32),
        pltpu.VMEM((_CH + 16,), jnp.int32),
        pltpu.VMEM((_CH, D_IN), jnp.float32),
        pltpu.VMEM((_NB + 16,), jnp.int32),
        pltpu.SemaphoreType.DMA,
    ],
)
def _agg(x_hbm, pkg_hbm, cnts_hbm, zacc_hbm, zcnt_hbm,
         agg_hbm, cnt_hbm,
         acc, cntacc, srcv, dstv, rows, cntv, sem):
    w = lax.axis_index("c") * 16 + lax.axis_index("s")
    lo = w * _BKN
    base = w * _CAP

    pltpu.sync_copy(zacc_hbm, acc)
    pltpu.sync_copy(zcnt_hbm, cntacc)
    pltpu.sync_copy(cnts_hbm, cntv)
    cb = cntv[pl.ds(w, 16)][0]
    nch = (cb + (_CH - 1)) // _CH

    def chunk(c, carry):
        off = base + c * _CH
        pltpu.sync_copy(pkg_hbm.at[pl.ds(off, _CH)], dstv.at[pl.ds(0, _CH)])
        for j in range(_CH // 16):
            sl = pl.ds(j * 16, 16)
            srcv[sl] = jnp.clip(dstv[sl] >> 14, 0, _NP - 1)
        pltpu.async_copy(x_hbm.at[srcv], rows, sem).wait()
        ne = jnp.minimum(cb - c * _CH, _CH)

        def edge(i, carry2):
            ld = jnp.clip((dstv[pl.ds(i, 16)][0] & 16383) - lo, 0, _BKN)
            cntacc[ld, :] = cntacc[ld, :] + 1.0
            for v in range(D_IN // 16):
                sl = pl.ds(v * 16, 16)
                acc[ld, sl] = acc[ld, sl] + rows[i, sl]
            return carry2

        lax.fori_loop(0, ne, edge, 0)
        return carry

    lax.fori_loop(0, nch, chunk, 0)

    pltpu.sync_copy(acc.at[pl.ds(0, _BKN)], agg_hbm.at[pl.ds(lo, _BKN)])
    pltpu.sync_copy(cntacc.at[pl.ds(0, _BKN)], cnt_hbm.at[pl.ds(lo, _BKN)])


_R1 = 1024
_R2 = 1000


def _dense1_body(agg_ref, c_ref, x_ref, wl_ref, b_ref, wr_ref, o_ref):
    cnt = jnp.clip(c_ref[:, 0:1], 1.0, None)
    agg = agg_ref[...] / cnt
    acc = jax.lax.dot(agg, wl_ref[...], precision=jax.lax.Precision.HIGHEST,
                      preferred_element_type=jnp.float32)
    acc += jax.lax.dot(x_ref[...], wr_ref[...], precision=jax.lax.Precision.HIGHEST,
                       preferred_element_type=jnp.float32)
    o_ref[...] = jnp.maximum(acc + b_ref[...], 0.0)


def _dense1(agg, cnt, x, wlT, b, wrT):
    return pl.pallas_call(
        _dense1_body,
        grid=(_NP // _R1,),
        in_specs=[
            pl.BlockSpec((_R1, D_IN), lambda i: (i, 0)),
            pl.BlockSpec((_R1, 16), lambda i: (i, 0)),
            pl.BlockSpec((_R1, D_IN), lambda i: (i, 0)),
            pl.BlockSpec((D_IN, D_HID), lambda i: (0, 0)),
            pl.BlockSpec((1, D_HID), lambda i: (0, 0)),
            pl.BlockSpec((D_IN, D_HID), lambda i: (0, 0)),
        ],
        out_specs=pl.BlockSpec((_R1, D_HID), lambda i: (i, 0)),
        out_shape=jax.ShapeDtypeStruct((_NP, D_HID), jnp.float32),
    )(agg, cnt, x, wlT, b, wrT)


def _dense2_body(p_ref, c_ref, h_ref, wl_ref, b_ref, wr_ref, wa_ref, ba_ref,
                 o_ref):
    cnt = jnp.clip(c_ref[:, 0:1], 1.0, None)
    agg = p_ref[...] / cnt
    acc = jax.lax.dot(agg, wl_ref[...], precision=jax.lax.Precision.HIGHEST,
                      preferred_element_type=jnp.float32)
    acc += jax.lax.dot(h_ref[...], wr_ref[...], precision=jax.lax.Precision.HIGHEST,
                       preferred_element_type=jnp.float32)
    h2 = jnp.maximum(acc + b_ref[...], 0.0)
    o_ref[...] = jax.lax.dot(h2, wa_ref[...], precision=jax.lax.Precision.HIGHEST,
                             preferred_element_type=jnp.float32) + ba_ref[...]


def _dense2(agg, cnt, h, wlT, b, wrT, waT, ba):
    return pl.pallas_call(
        _dense2_body,
        grid=(N // _R2,),
        in_specs=[
            pl.BlockSpec((_R2, D_HID), lambda i: (i, 0)),
            pl.BlockSpec((_R2, 16), lambda i: (i, 0)),
            pl.BlockSpec((_R2, D_HID), lambda i: (i, 0)),
            pl.BlockSpec((D_HID, D_HID), lambda i: (0, 0)),
            pl.BlockSpec((1, D_HID), lambda i: (0, 0)),
            pl.BlockSpec((D_HID, D_HID), lambda i: (0, 0)),
            pl.BlockSpec((D_HID, D_OUT), lambda i: (0, 0)),
            pl.BlockSpec((1, D_OUT), lambda i: (0, 0)),
        ],
        out_specs=pl.BlockSpec((_R2, D_OUT), lambda i: (i, 0)),
        out_shape=jax.ShapeDtypeStruct((N, D_OUT), jnp.float32),
    )(agg, cnt, h, wlT, b, wrT, waT, ba)


def kernel(x, edge_index, W1_l, b1, W1_r, W2_l, b2, W2_r, Wa, ba):
    src = edge_index[0].astype(jnp.int32)
    dst = edge_index[1].astype(jnp.int32)

    tri = jnp.tril(jnp.ones((_C, _C), jnp.float32), -1)
    dstp = jnp.pad(dst, (0, _NCHP * _C - E)).reshape(_NCHP, _C)
    pos2d, cntf = _rank(dstp, tri)
    pos = pos2d.reshape(_NCHP * _C)[:E]
    cnts = jnp.pad(cntf[0].astype(jnp.int32), (0, 16))

    pkg = _group(src, dst, pos)

    xpad = jnp.pad(x, ((0, _NP - N), (0, 0)))
    zacc = jnp.zeros((_BKN + 8, D_IN), jnp.float32)
    zcnt = jnp.zeros((_BKN + 8, 16), jnp.float32)

    agg1, cnt = _agg(xpad, pkg, cnts, zacc, zcnt)
    h1 = _dense1(agg1, cnt, xpad, W1_l.T, b1[None, :], W1_r.T)
    agg2, _ = _agg(h1, pkg, cnts, zacc, zcnt)
    return _dense2(agg2, cnt, h1, W2_l.T, b2[None, :], W2_r.T, Wa.T, ba[None, :])


# speedup vs baseline: 1.4285x; 1.0213x over previous
import functools

import jax
import jax.numpy as jnp
from jax import lax
from jax.experimental import pallas as pl
from jax.experimental.pallas import tpu as pltpu
from jax.experimental.pallas import tpu_sc as plsc

N = 10000
E = 320000
D_IN = 128
D_HID = 128
D_OUT = 64

_NP = 10240
_NB = 32
_BKN = _NP // _NB
_CAP = E
_CH = 80
_C = 512
_NCH = E // _C


def _rank_body(dst_ref, src_ref, tri_ref, pos_ref, pk_ref, cnt_ref, carry):
    c = pl.program_id(0)

    @pl.when(c == 0)
    def _():
        carry[...] = jnp.zeros_like(carry)

    for r in range(8):
        d = dst_ref[r]
        b = d // _BKN
        real = (c * 8 + r < _NCH).astype(jnp.float32)
        oh = (b[:, None] == lax.broadcasted_iota(jnp.int32, (_C, _NB), 1)
              ).astype(jnp.float32) * real
        pr = jax.lax.dot(tri_ref[...], oh, precision=jax.lax.Precision.HIGHEST,
                         preferred_element_type=jnp.float32)
        rank = jnp.sum(pr * oh, axis=1)
        pg = jnp.sum(carry[0][None, :] * oh, axis=1)
        pos_ref[r, :] = b * _CAP + (rank + pg).astype(jnp.int32)
        pk_ref[r, :] = (src_ref[r] << 14) | d
        carry[...] = carry[...] + jnp.sum(oh, axis=0)[None]
    cnt_ref[...] = carry[...]


_NCHP = 640


def _rank(dst2d, src2d, tri):
    return pl.pallas_call(
        _rank_body,
        grid=(_NCHP // 8,),
        in_specs=[
            pl.BlockSpec((8, _C), lambda i: (i, 0)),
            pl.BlockSpec((8, _C), lambda i: (i, 0)),
            pl.BlockSpec((_C, _C), lambda i: (0, 0)),
        ],
        out_specs=[
            pl.BlockSpec((8, _C), lambda i: (i, 0)),
            pl.BlockSpec((8, _C), lambda i: (i, 0)),
            pl.BlockSpec((1, _NB), lambda i: (0, 0)),
        ],
        out_shape=[
            jax.ShapeDtypeStruct((_NCHP, _C), jnp.int32),
            jax.ShapeDtypeStruct((_NCHP, _C), jnp.int32),
            jax.ShapeDtypeStruct((1, _NB), jnp.float32),
        ],
        scratch_shapes=[pltpu.VMEM((1, _NB), jnp.float32)],
    )(dst2d, src2d, tri)


_mesh = plsc.VectorSubcoreMesh(core_axis_name="c", subcore_axis_name="s")
_EPT = E // _NB


@functools.partial(
    pl.kernel,
    out_type=jax.ShapeDtypeStruct((_NB * _CAP,), jnp.int32),
    mesh=_mesh,
    scratch_types=[
        pltpu.VMEM((_CH,), jnp.int32),
        pltpu.VMEM((_CH,), jnp.int32),
    ],
)
def _group(pk_hbm, pos_hbm, pkg_hbm, posv, pkv):
    w = lax.axis_index("c") * 16 + lax.axis_index("s")
    base = w * _EPT

    def chunk(c, carry):
        off = base + c * _CH
        pltpu.sync_copy(pk_hbm.at[pl.ds(off, _CH)], pkv)
        pltpu.sync_copy(pos_hbm.at[pl.ds(off, _CH)], posv)
        pltpu.sync_copy(pkv, pkg_hbm.at[posv])
        return carry

    lax.fori_loop(0, _EPT // _CH, chunk, 0)


def _make_agg(with_cnt):
    outs = jax.ShapeDtypeStruct((_NP, D_IN), jnp.float32)
    if with_cnt:
        outs = (outs, jax.ShapeDtypeStruct((_NP, 16), jnp.float32))
    scratch = [pltpu.VMEM((_BKN + 8, D_IN), jnp.float32)]
    if with_cnt:
        scratch.append(pltpu.VMEM((_BKN + 8, 16), jnp.float32))
    scratch += [
        pltpu.VMEM((_CH,), jnp.int32),
        pltpu.VMEM((_CH + 16,), jnp.int32),
        pltpu.VMEM((_CH, D_IN), jnp.float32),
        pltpu.VMEM((_NB + 16,), jnp.int32),
        pltpu.SemaphoreType.DMA,
    ]

    @functools.partial(pl.kernel, out_type=outs, mesh=_mesh,
                       scratch_types=scratch)
    def _agg(*refs):
        if with_cnt:
            (x_hbm, pkg_hbm, cnts_hbm, zacc_hbm, zcnt_hbm,
             agg_hbm, cnt_hbm,
             acc, cntacc, srcv, dstv, rows, cntv, sem) = refs
        else:
            (x_hbm, pkg_hbm, cnts_hbm, zacc_hbm,
             agg_hbm,
             acc, srcv, dstv, rows, cntv, sem) = refs
        w = lax.axis_index("c") * 16 + lax.axis_index("s")
        lo = w * _BKN
        base = w * _CAP

        pltpu.sync_copy(zacc_hbm, acc)
        if with_cnt:
            pltpu.sync_copy(zcnt_hbm, cntacc)
        pltpu.sync_copy(cnts_hbm, cntv)
        cb = cntv[pl.ds(w, 16)][0]
        nch = (cb + (_CH - 1)) // _CH

        def chunk(c, carry):
            off = base + c * _CH
            pltpu.sync_copy(pkg_hbm.at[pl.ds(off, _CH)], dstv.at[pl.ds(0, _CH)])
            for j in range(_CH // 16):
                sl = pl.ds(j * 16, 16)
                srcv[sl] = jnp.clip(dstv[sl] >> 14, 0, _NP - 1)
            pltpu.async_copy(x_hbm.at[srcv], rows, sem).wait()
            ne = jnp.minimum(cb - c * _CH, _CH)

            def edge(i, carry2):
                ld = jnp.clip((dstv[pl.ds(i, 16)][0] & 16383) - lo, 0, _BKN)
                if with_cnt:
                    cntacc[ld, :] = cntacc[ld, :] + 1.0
                for v in range(D_IN // 16):
                    sl = pl.ds(v * 16, 16)
                    acc[ld, sl] = acc[ld, sl] + rows[i, sl]
                return carry2

            lax.fori_loop(0, ne, edge, 0)
            return carry

        lax.fori_loop(0, nch, chunk, 0)

        pltpu.sync_copy(acc.at[pl.ds(0, _BKN)], agg_hbm.at[pl.ds(lo, _BKN)])
        if with_cnt:
            pltpu.sync_copy(cntacc.at[pl.ds(0, _BKN)],
                            cnt_hbm.at[pl.ds(lo, _BKN)])

    return _agg


_agg_cnt = _make_agg(True)
_agg_nocnt = _make_agg(False)


_R1 = 1024
_R2 = 1000


def _dense1_body(agg_ref, c_ref, x_ref, wl_ref, b_ref, wr_ref, o_ref):
    cnt = jnp.clip(c_ref[:, 0:1], 1.0, None)
    agg = agg_ref[...] / cnt
    acc = jax.lax.dot(agg, wl_ref[...], precision=jax.lax.Precision.HIGHEST,
                      preferred_element_type=jnp.float32)
    acc += jax.lax.dot(x_ref[...], wr_ref[...], precision=jax.lax.Precision.HIGHEST,
                       preferred_element_type=jnp.float32)
    o_ref[...] = jnp.maximum(acc + b_ref[...], 0.0)


def _dense1(agg, cnt, x, wlT, b, wrT):
    return pl.pallas_call(
        _dense1_body,
        grid=(_NP // _R1,),
        in_specs=[
            pl.BlockSpec((_R1, D_IN), lambda i: (i, 0)),
            pl.BlockSpec((_R1, 16), lambda i: (i, 0)),
            pl.BlockSpec((_R1, D_IN), lambda i: (i, 0)),
            pl.BlockSpec((D_IN, D_HID), lambda i: (0, 0)),
            pl.BlockSpec((1, D_HID), lambda i: (0, 0)),
            pl.BlockSpec((D_IN, D_HID), lambda i: (0, 0)),
        ],
        out_specs=pl.BlockSpec((_R1, D_HID), lambda i: (i, 0)),
        out_shape=jax.ShapeDtypeStruct((_NP, D_HID), jnp.float32),
    )(agg, cnt, x, wlT, b, wrT)


def _dense2_body(p_ref, c_ref, h_ref, wl_ref, b_ref, wr_ref, wa_ref, ba_ref,
                 o_ref):
    cnt = jnp.clip(c_ref[:, 0:1], 1.0, None)
    agg = p_ref[...] / cnt
    acc = jax.lax.dot(agg, wl_ref[...], precision=jax.lax.Precision.HIGHEST,
                      preferred_element_type=jnp.float32)
    acc += jax.lax.dot(h_ref[...], wr_ref[...], precision=jax.lax.Precision.HIGHEST,
                       preferred_element_type=jnp.float32)
    h2 = jnp.maximum(acc + b_ref[...], 0.0)
    o_ref[...] = jax.lax.dot(h2, wa_ref[...], precision=jax.lax.Precision.HIGHEST,
                             preferred_element_type=jnp.float32) + ba_ref[...]


def _dense2(agg, cnt, h, wlT, b, wrT, waT, ba):
    return pl.pallas_call(
        _dense2_body,
        grid=(N // _R2,),
        in_specs=[
            pl.BlockSpec((_R2, D_HID), lambda i: (i, 0)),
            pl.BlockSpec((_R2, 16), lambda i: (i, 0)),
            pl.BlockSpec((_R2, D_HID), lambda i: (i, 0)),
            pl.BlockSpec((D_HID, D_HID), lambda i: (0, 0)),
            pl.BlockSpec((1, D_HID), lambda i: (0, 0)),
            pl.BlockSpec((D_HID, D_HID), lambda i: (0, 0)),
            pl.BlockSpec((D_HID, D_OUT), lambda i: (0, 0)),
            pl.BlockSpec((1, D_OUT), lambda i: (0, 0)),
        ],
        out_specs=pl.BlockSpec((_R2, D_OUT), lambda i: (i, 0)),
        out_shape=jax.ShapeDtypeStruct((N, D_OUT), jnp.float32),
    )(agg, cnt, h, wlT, b, wrT, waT, ba)


def kernel(x, edge_index, W1_l, b1, W1_r, W2_l, b2, W2_r, Wa, ba):
    src = edge_index[0].astype(jnp.int32)
    dst = edge_index[1].astype(jnp.int32)

    tri = jnp.tril(jnp.ones((_C, _C), jnp.float32), -1)
    dstp = jnp.pad(dst, (0, _NCHP * _C - E)).reshape(_NCHP, _C)
    srcp = jnp.pad(src, (0, _NCHP * _C - E)).reshape(_NCHP, _C)
    pos2d, pk2d, cntf = _rank(dstp, srcp, tri)
    pos = pos2d.reshape(_NCHP * _C)[:E]
    pk = pk2d.reshape(_NCHP * _C)[:E]
    cnts = jnp.pad(cntf[0].astype(jnp.int32), (0, 16))

    pkg = _group(pk, pos)

    xpad = jnp.pad(x, ((0, _NP - N), (0, 0)))
    zacc = jnp.zeros((_BKN + 8, D_IN), jnp.float32)
    zcnt = jnp.zeros((_BKN + 8, 16), jnp.float32)

    agg1, cnt = _agg_cnt(xpad, pkg, cnts, zacc, zcnt)
    h1 = _dense1(agg1, cnt, xpad, W1_l.T, b1[None, :], W1_r.T)
    agg2 = _agg_nocnt(h1, pkg, cnts, zacc)
    return _dense2(agg2, cnt, h1, W2_l.T, b2[None, :], W2_r.T, Wa.T, ba[None, :])
